# baseline (speedup 1.0000x reference)
import functools
import jax
import jax.numpy as jnp
from jax import lax
from jax.experimental import pallas as pl
from jax.experimental.pallas import tpu as pltpu
from jax.experimental.pallas import tpu_sc as plsc

_NC, _NS, _L = 2, 16, 16
_NW = _NC * _NS
_RC = 512 // (_NW // 8)
_SUB = 32
_OUT_T = jax.ShapeDtypeStruct((8, 512, 8, 1024), jnp.float32)


def _extract(chunks, p):
    lane = jnp.full((_L,), p % _L, jnp.int32)
    m = lax.iota(jnp.int32, _L) == lane
    c = p // _L
    v = chunks[-1]
    for cc in range(len(chunks) - 2, -1, -1):
        v = jnp.where(c == cc, chunks[cc], v)
    return jnp.max(jnp.where(m, v, jnp.int32(-1)))


def _tile_prelude(plan_hbm, plan_v):
    wid = lax.axis_index("s") * _NC + lax.axis_index("c")
    r0 = (wid // 8) * _RC
    g = (wid % 8) * 8
    pltpu.sync_copy(plan_hbm, plan_v)
    f_chunks = [plan_v[pl.ds(c * _L, _L)] for c in range(4)]
    i_chunks = [plan_v[pl.ds(64 + c * _L, _L)] for c in range(4)]
    j_chunks = [plan_v[pl.ds(128 + c * _L, _L)] for c in range(4)]
    fs = [_extract(f_chunks, g + k) for k in range(8)]
    is_ = [_extract(i_chunks, g + k) for k in range(8)]
    js = [_extract(j_chunks, g + k) for k in range(8)]
    return r0, fs, is_, js


def _sc_body_general(params_hbm, plan_hbm, out_hbm, plan_v, buf, sem_w):
    del sem_w
    r0, fs, is_, js = _tile_prelude(plan_hbm, plan_v)
    for s in range(_RC // _SUB):
        r0s = r0 + s * _SUB
        for k in range(8):
            def _load(k=k, r0s=r0s):
                pltpu.sync_copy(
                    params_hbm.at[fs[k], pl.ds(r0s, _SUB), :], buf
                )

            if k == 0:
                _load()
            else:
                pl.when(fs[k] != fs[k - 1])(_load)

            pltpu.sync_copy(
                buf, out_hbm.at[is_[k], pl.ds(r0s, _SUB), js[k], :]
            )


def _sc_body_fast(params_hbm, plan_hbm, out_hbm, plan_v, buf2, sem_l, sem_w):
    r0, fs, is_, js = _tile_prelude(plan_hbm, plan_v)
    n_half = 2 * (_RC // _SUB)

    def _start_load(h):
        s, khalf = divmod(h, 2)
        r0s = r0 + s * _SUB
        return pltpu.async_copy(
            params_hbm.at[fs[khalf * 4], pl.ds(r0s, _SUB), :],
            buf2.at[h % 2],
            sem_l,
        )

    loads = {0: _start_load(0)}
    writes = {}
    for h in range(n_half):
        loads.pop(h).wait()
        if h + 1 < n_half:
            for w in writes.pop(h - 1, ()):
                w.wait()
            loads[h + 1] = _start_load(h + 1)
        s, khalf = divmod(h, 2)
        r0s = r0 + s * _SUB
        writes[h] = [
            pltpu.async_copy(
                buf2.at[h % 2],
                out_hbm.at[is_[khalf * 4 + kk], pl.ds(r0s, _SUB),
                           js[khalf * 4 + kk], :],
                sem_w,
            )
            for kk in range(4)
        ]
    for hs in writes.values():
        for w in hs:
            w.wait()


_MESH = plsc.VectorSubcoreMesh(core_axis_name="c", subcore_axis_name="s")
_CPARAMS = pltpu.CompilerParams(needs_layout_passes=False)

_general_kernel = functools.partial(
    pl.kernel,
    out_type=_OUT_T,
    mesh=_MESH,
    scratch_types=[
        pltpu.VMEM((192,), jnp.int32),
        pltpu.VMEM((_SUB, 1024), jnp.float32),
        pltpu.SemaphoreType.DMA,
    ],
    compiler_params=_CPARAMS,
)(_sc_body_general)

_fast_kernel = functools.partial(
    pl.kernel,
    out_type=_OUT_T,
    mesh=_MESH,
    scratch_types=[
        pltpu.VMEM((192,), jnp.int32),
        pltpu.VMEM((2, _SUB, 1024), jnp.float32),
        pltpu.SemaphoreType.DMA,
        pltpu.SemaphoreType.DMA,
    ],
    compiler_params=_CPARAMS,
)(_sc_body_fast)


def kernel(params, perm_index):
    flat = perm_index.reshape(64).astype(jnp.int32)
    order = jnp.argsort(flat).astype(jnp.int32)
    f_s = jnp.take(flat, order)
    plan = jnp.concatenate([f_s, order // 8, order % 8]).astype(jnp.int32)
    grp = f_s.reshape(16, 4)
    aligned = jnp.all(grp == grp[:, :1])
    return lax.cond(aligned, _fast_kernel, _general_kernel, params, plan)
